# baseline (speedup 1.0000x reference)
import functools

import jax
import jax.numpy as jnp
from jax import lax
from jax.experimental import pallas as pl
from jax.experimental.pallas import tpu as pltpu, tpu_sc as plsc

_B = 4
_CIN = 19
_COUT = 34
_H, _W = 512, 1024
_ZERO_VAL = -1000000.0
_IDS = (7, 8, 11, 12, 13, 17, 19, 20, 21, 22, 23, 24, 25, 26, 27, 28, 31, 32, 33)
_FILL = tuple(c for c in range(_COUT) if c not in _IDS)

_NC, _NS = 2, 16
_NW = _NC * _NS
_RPW = _H // _NW


_NBUF = 6
_N_FILL_DMA = 0


def _body(in_hbm, out_hbm, fill_ref, bufs, fill_sem, gsems, ssems):
    wid = lax.axis_index("s") * _NC + lax.axis_index("c")
    row0 = wid * _RPW

    neg = jnp.full((16,), _ZERO_VAL, dtype=jnp.float32)

    def _fill_row(i, _):
        for j in range(_W // 16):
            fill_ref[i, pl.ds(j * 16, 16)] = neg
        return 0

    lax.fori_loop(0, _RPW, _fill_row, 0)

    fill_slabs = [b * _COUT + c for b in range(_B) for c in _FILL]
    fill_handles = []
    for slab in fill_slabs:
        fill_handles.append(pltpu.async_copy(
            fill_ref,
            out_hbm.at[slab, pl.ds(row0, _RPW)],
            fill_sem,
        ))

    copies = [(b * _CIN + t, b * _COUT + c)
              for b in range(_B) for t, c in enumerate(_IDS)]
    n = len(copies)
    gather_h = [None] * n
    scatter_h = [None] * n
    gd = 0
    sd = 0
    for i in range(n + 1):
        if i < n:
            slot = i % _NBUF
            if i >= _NBUF:
                target = min(i - _NBUF + 2, i - 1)
                while sd < target:
                    scatter_h[sd].wait()
                    sd += 1
            gather_h[i] = pltpu.async_copy(
                in_hbm.at[copies[i][0], pl.ds(row0, _RPW)],
                bufs[slot], gsems[slot])
        if i >= 1:
            j = i - 1
            target = min(j + 2, n)
            while gd < target:
                gather_h[gd].wait()
                gd += 1
            scatter_h[j] = pltpu.async_copy(
                bufs[j % _NBUF],
                out_hbm.at[copies[j][1], pl.ds(row0, _RPW)],
                ssems[j % _NBUF])
    while sd < n:
        scatter_h[sd].wait()
        sd += 1
    for h in fill_handles:
        h.wait()


@jax.jit
def kernel(prediction):
    flat_in = prediction.reshape(_B * _CIN, _H, _W)
    mesh = plsc.VectorSubcoreMesh(core_axis_name="c", subcore_axis_name="s")
    k = functools.partial(
        pl.kernel,
        mesh=mesh,
        out_type=jax.ShapeDtypeStruct((_B * _COUT, _H, _W), jnp.float32),
        scratch_types=[
            pltpu.VMEM((_RPW, _W), jnp.float32),
            [pltpu.VMEM((_RPW, _W), jnp.float32) for _ in range(_NBUF)],
            pltpu.SemaphoreType.DMA,
            [pltpu.SemaphoreType.DMA for _ in range(_NBUF)],
            [pltpu.SemaphoreType.DMA for _ in range(_NBUF)],
        ],
    )(_body)
    out = k(flat_in)
    return out.reshape(_B, _COUT, _H, _W)


# speedup vs baseline: 1.1869x; 1.1869x over previous
import functools

import jax
import jax.numpy as jnp
from jax import lax
from jax.experimental import pallas as pl
from jax.experimental.pallas import tpu as pltpu, tpu_sc as plsc

_B = 4
_CIN = 19
_COUT = 34
_H, _W = 512, 1024
_ZERO_VAL = -1000000.0
_IDS = (7, 8, 11, 12, 13, 17, 19, 20, 21, 22, 23, 24, 25, 26, 27, 28, 31, 32, 33)
_FILL = tuple(c for c in range(_COUT) if c not in _IDS)

_NC, _NS = 2, 16
_NW = _NC * _NS
_RPW = _H // _NW


_NBUF = 6
_N_FILL_DMA = 0


def _body(in_hbm, out_hbm, fill_ref, bufs, fill_sem, gsems, ssems):
    wid = lax.axis_index("s") * _NC + lax.axis_index("c")
    row0 = wid * _RPW

    neg = jnp.full((16,), _ZERO_VAL, dtype=jnp.float32)

    def _fill_row(i, _):
        for j in range(_W // 16):
            fill_ref[i, pl.ds(j * 16, 16)] = neg
        return 0

    lax.fori_loop(0, _RPW, _fill_row, 0)

    fill_slabs = [b * _COUT + c for b in range(_B) for c in _FILL]
    fill_handles = []
    for slab in fill_slabs:
        fill_handles.append(pltpu.async_copy(
            fill_ref,
            out_hbm.at[slab, pl.ds(row0, _RPW)],
            fill_sem,
        ))

    copies = [(b * _CIN + t, b * _COUT + c)
              for b in range(_B) for t, c in enumerate(_IDS)]
    n = len(copies)
    gather_h = [None] * _NBUF
    scatter_h = [None] * n
    sd = 0
    for i in range(n + 1):
        if i < n:
            slot = i % _NBUF
            if i >= _NBUF:
                target = min(i - _NBUF + 2, i - 1)
                while sd < target:
                    scatter_h[sd].wait()
                    sd += 1
            gather_h[slot] = pltpu.async_copy(
                in_hbm.at[copies[i][0], pl.ds(row0, _RPW)],
                bufs[slot], gsems[slot])
        if i >= 1:
            j = i - 1
            slot = j % _NBUF
            gather_h[slot].wait()
            scatter_h[j] = pltpu.async_copy(
                bufs[slot],
                out_hbm.at[copies[j][1], pl.ds(row0, _RPW)],
                ssems[slot])
    while sd < n:
        scatter_h[sd].wait()
        sd += 1
    for h in fill_handles:
        h.wait()


@jax.jit
def kernel(prediction):
    flat_in = prediction.reshape(_B * _CIN, _H, _W)
    mesh = plsc.VectorSubcoreMesh(core_axis_name="c", subcore_axis_name="s")
    k = functools.partial(
        pl.kernel,
        mesh=mesh,
        out_type=jax.ShapeDtypeStruct((_B * _COUT, _H, _W), jnp.float32),
        scratch_types=[
            pltpu.VMEM((_RPW, _W), jnp.float32),
            [pltpu.VMEM((_RPW, _W), jnp.float32) for _ in range(_NBUF)],
            pltpu.SemaphoreType.DMA,
            [pltpu.SemaphoreType.DMA for _ in range(_NBUF)],
            [pltpu.SemaphoreType.DMA for _ in range(_NBUF)],
        ],
    )(_body)
    out = k(flat_in)
    return out.reshape(_B, _COUT, _H, _W)
